# baseline (speedup 1.0000x reference)
import functools

import jax
import jax.numpy as jnp
from jax import lax
from jax.experimental import pallas as pl
from jax.experimental.pallas import tpu as pltpu
from jax.experimental.pallas import tpu_sc as plsc

N_NODES = 10000
N_EDGES = 320000
D_FEAT = 128
L = 16
NC, NS = 2, 16
NW = NC * NS

TILE = 128
N_TILES = N_EDGES // TILE
T_PER_W = N_TILES // NW
CHUNK = T_PER_W * TILE
REM_BASE = NW * CHUNK
N_REM = N_TILES - NW * T_PER_W
HALF = CHUNK // 2


_TC_CHUNK = 1024
_TC_SIZES = [_TC_CHUNK] * 9 + [N_NODES - 9 * _TC_CHUNK]
_TC_OFFS = [i * _TC_CHUNK for i in range(10)]
_N_PAD = 10 * _TC_CHUNK
_N_SEM = 4


def _tc_node_scores(b_ref, w_ref, h_hbm, o_ref, hbuf, acc, *sems):
    def _copy(i):
        return pltpu.make_async_copy(
            h_hbm.at[pl.ds(_TC_OFFS[i], _TC_SIZES[i])],
            hbuf.at[pl.ds(_TC_OFFS[i], _TC_SIZES[i])],
            sems[i % _N_SEM],
        )

    for i in range(len(_TC_SIZES)):
        _copy(i).start()
    w = w_ref[...]
    for i in range(len(_TC_SIZES)):
        _copy(i).wait()
        hb = hbuf[pl.ds(_TC_OFFS[i], _TC_SIZES[i]), :]
        acc[:, pl.ds(_TC_OFFS[i], _TC_SIZES[i])] = lax.dot_general(
            w, hb,
            dimension_numbers=(((1,), (1,)), ((), ())),
            preferred_element_type=jnp.float32,
        )
    rows = lax.broadcasted_iota(jnp.int32, (2, N_NODES), 0)
    bias = jnp.where(rows == 1, b_ref[0, 0], 0.0)
    o_ref[...] = acc[:, pl.ds(0, N_NODES)] + bias


_sc_mesh = plsc.VectorSubcoreMesh(core_axis_name="c", subcore_axis_name="s")


@functools.partial(
    pl.kernel,
    mesh=_sc_mesh,
    out_type=jax.ShapeDtypeStruct((1, N_EDGES), jnp.float32),
    compiler_params=pltpu.CompilerParams(needs_layout_passes=False),
    scratch_types=[
        pltpu.VMEM((N_NODES,), jnp.float32),
        pltpu.VMEM((N_NODES,), jnp.float32),
        pltpu.VMEM((2, CHUNK), jnp.int32),
        pltpu.VMEM((2, TILE), jnp.int32),
        pltpu.VMEM((CHUNK,), jnp.float32),
        pltpu.VMEM((TILE,), jnp.float32),
        pltpu.SemaphoreType.DMA,
        pltpu.SemaphoreType.DMA,
        pltpu.SemaphoreType.DMA,
    ],
)
def _sc_edge_gather(s_hbm, edge_hbm, out_hbm,
                    s1_v, s2_v, ei_v, ei2_v, out_v, out2_v,
                    sem, sem_b, sem_out):
    wid = lax.axis_index("s") * NC + lax.axis_index("c")
    base = wid * CHUNK
    rbase = REM_BASE + wid * TILE
    c1 = pltpu.async_copy(s_hbm.at[0], s1_v, sem)
    c2 = pltpu.async_copy(s_hbm.at[1], s2_v, sem)
    ca = pltpu.async_copy(edge_hbm.at[:, pl.ds(base, HALF)],
                          ei_v.at[:, pl.ds(0, HALF)], sem)
    cb = pltpu.async_copy(edge_hbm.at[:, pl.ds(base + HALF, HALF)],
                          ei_v.at[:, pl.ds(HALF, HALF)], sem_b)

    @pl.when(wid < N_REM)
    def _rem_fetch():
        pltpu.async_copy(edge_hbm.at[:, pl.ds(rbase, TILE)], ei2_v, sem_b)

    c1.wait()
    c2.wait()
    ca.wait()

    @plsc.parallel_loop(0, HALF, L, unroll=8)
    def _body_a(off):
        si = ei_v[0, pl.ds(off, L)]
        di = ei_v[1, pl.ds(off, L)]
        v1 = plsc.load_gather(s1_v, [si])
        v2 = plsc.load_gather(s2_v, [di])
        out_v[pl.ds(off, L)] = v1 + v2

    oa = pltpu.async_copy(out_v.at[pl.ds(0, HALF)],
                          out_hbm.at[0, pl.ds(base, HALF)], sem_out)
    cb.wait()

    @plsc.parallel_loop(HALF, CHUNK, L, unroll=8)
    def _body_b(off):
        si = ei_v[0, pl.ds(off, L)]
        di = ei_v[1, pl.ds(off, L)]
        v1 = plsc.load_gather(s1_v, [si])
        v2 = plsc.load_gather(s2_v, [di])
        out_v[pl.ds(off, L)] = v1 + v2

    ob = pltpu.async_copy(out_v.at[pl.ds(HALF, HALF)],
                          out_hbm.at[0, pl.ds(base + HALF, HALF)], sem_out)

    @pl.when(wid < N_REM)
    def _rem():
        pltpu.make_async_copy(edge_hbm.at[:, pl.ds(rbase, TILE)],
                              ei2_v, sem_b).wait()

        @plsc.parallel_loop(0, TILE, L, unroll=8)
        def _body2(off):
            si = ei2_v[0, pl.ds(off, L)]
            di = ei2_v[1, pl.ds(off, L)]
            v1 = plsc.load_gather(s1_v, [si])
            v2 = plsc.load_gather(s2_v, [di])
            out2_v[pl.ds(off, L)] = v1 + v2

        pltpu.sync_copy(out2_v, out_hbm.at[0, pl.ds(rbase, TILE)])

    oa.wait()
    ob.wait()


def kernel(h, edge_index, W, b):
    wmat = W.reshape(2, D_FEAT)
    s = pl.pallas_call(
        _tc_node_scores,
        in_specs=[
            pl.BlockSpec(memory_space=pltpu.SMEM),
            pl.BlockSpec((2, D_FEAT)),
            pl.BlockSpec(memory_space=pl.ANY),
        ],
        out_specs=pl.BlockSpec((2, N_NODES)),
        out_shape=jax.ShapeDtypeStruct((2, N_NODES), jnp.float32),
        scratch_shapes=[
            pltpu.VMEM((_N_PAD, D_FEAT), jnp.float32),
            pltpu.VMEM((2, _N_PAD), jnp.float32),
        ] + [pltpu.SemaphoreType.DMA] * _N_SEM,
    )(b.reshape(1, 1), wmat, h)
    scores = _sc_edge_gather(s, edge_index)
    return scores.reshape(N_EDGES, 1)


# speedup vs baseline: 1.0055x; 1.0055x over previous
import functools

import jax
import jax.numpy as jnp
from jax import lax
from jax.experimental import pallas as pl
from jax.experimental.pallas import tpu as pltpu
from jax.experimental.pallas import tpu_sc as plsc

N_NODES = 10000
N_EDGES = 320000
D_FEAT = 128
L = 16
NC, NS = 2, 16
NW = NC * NS

TILE = 128
N_TILES = N_EDGES // TILE
T_PER_W = N_TILES // NW
CHUNK = T_PER_W * TILE
REM_BASE = NW * CHUNK
N_REM = N_TILES - NW * T_PER_W
HALF = CHUNK // 2


def _tc_node_scores(b_ref, w_ref, h_ref, o_ref):
    rows = lax.broadcasted_iota(jnp.int32, (2, N_NODES), 0)
    bias = jnp.where(rows == 1, b_ref[0, 0], 0.0)
    o_ref[...] = (
        lax.dot_general(
            w_ref[...], h_ref[...],
            dimension_numbers=(((1,), (1,)), ((), ())),
            preferred_element_type=jnp.float32,
        )
        + bias
    )


_sc_mesh = plsc.VectorSubcoreMesh(core_axis_name="c", subcore_axis_name="s")


@functools.partial(
    pl.kernel,
    mesh=_sc_mesh,
    out_type=jax.ShapeDtypeStruct((1, N_EDGES), jnp.float32),
    compiler_params=pltpu.CompilerParams(needs_layout_passes=False),
    scratch_types=[
        pltpu.VMEM((N_NODES,), jnp.float32),
        pltpu.VMEM((N_NODES,), jnp.float32),
        pltpu.VMEM((2, CHUNK), jnp.int32),
        pltpu.VMEM((2, TILE), jnp.int32),
        pltpu.VMEM((CHUNK,), jnp.float32),
        pltpu.VMEM((TILE,), jnp.float32),
        pltpu.SemaphoreType.DMA,
        pltpu.SemaphoreType.DMA,
        pltpu.SemaphoreType.DMA,
    ],
)
def _sc_edge_gather(s_hbm, edge_hbm, out_hbm,
                    s1_v, s2_v, ei_v, ei2_v, out_v, out2_v,
                    sem, sem_b, sem_out):
    wid = lax.axis_index("s") * NC + lax.axis_index("c")
    base = wid * CHUNK
    rbase = REM_BASE + wid * TILE
    c1 = pltpu.async_copy(s_hbm.at[0], s1_v, sem)
    c2 = pltpu.async_copy(s_hbm.at[1], s2_v, sem)
    ca = pltpu.async_copy(edge_hbm.at[:, pl.ds(base, HALF)],
                          ei_v.at[:, pl.ds(0, HALF)], sem)
    cb = pltpu.async_copy(edge_hbm.at[:, pl.ds(base + HALF, HALF)],
                          ei_v.at[:, pl.ds(HALF, HALF)], sem_b)

    @pl.when(wid < N_REM)
    def _rem_fetch():
        pltpu.async_copy(edge_hbm.at[:, pl.ds(rbase, TILE)], ei2_v, sem_b)

    c1.wait()
    c2.wait()
    ca.wait()

    @plsc.parallel_loop(0, HALF, L, unroll=24)
    def _body_a(off):
        si = ei_v[0, pl.ds(off, L)]
        di = ei_v[1, pl.ds(off, L)]
        v1 = plsc.load_gather(s1_v, [si])
        v2 = plsc.load_gather(s2_v, [di])
        out_v[pl.ds(off, L)] = v1 + v2

    oa = pltpu.async_copy(out_v.at[pl.ds(0, HALF)],
                          out_hbm.at[0, pl.ds(base, HALF)], sem_out)
    cb.wait()

    @plsc.parallel_loop(HALF, CHUNK, L, unroll=24)
    def _body_b(off):
        si = ei_v[0, pl.ds(off, L)]
        di = ei_v[1, pl.ds(off, L)]
        v1 = plsc.load_gather(s1_v, [si])
        v2 = plsc.load_gather(s2_v, [di])
        out_v[pl.ds(off, L)] = v1 + v2

    ob = pltpu.async_copy(out_v.at[pl.ds(HALF, HALF)],
                          out_hbm.at[0, pl.ds(base + HALF, HALF)], sem_out)

    @pl.when(wid < N_REM)
    def _rem():
        pltpu.make_async_copy(edge_hbm.at[:, pl.ds(rbase, TILE)],
                              ei2_v, sem_b).wait()

        @plsc.parallel_loop(0, TILE, L, unroll=8)
        def _body2(off):
            si = ei2_v[0, pl.ds(off, L)]
            di = ei2_v[1, pl.ds(off, L)]
            v1 = plsc.load_gather(s1_v, [si])
            v2 = plsc.load_gather(s2_v, [di])
            out2_v[pl.ds(off, L)] = v1 + v2

        pltpu.sync_copy(out2_v, out_hbm.at[0, pl.ds(rbase, TILE)])

    oa.wait()
    ob.wait()


def kernel(h, edge_index, W, b):
    wmat = W.reshape(2, D_FEAT)
    s = pl.pallas_call(
        _tc_node_scores,
        in_specs=[
            pl.BlockSpec(memory_space=pltpu.SMEM),
            pl.BlockSpec((2, D_FEAT)),
            pl.BlockSpec((N_NODES, D_FEAT)),
        ],
        out_specs=pl.BlockSpec((2, N_NODES)),
        out_shape=jax.ShapeDtypeStruct((2, N_NODES), jnp.float32),
    )(b.reshape(1, 1), wmat, h)
    scores = _sc_edge_gather(s, edge_index)
    return scores.reshape(N_EDGES, 1)


# speedup vs baseline: 1.0302x; 1.0245x over previous
import functools

import jax
import jax.numpy as jnp
from jax import lax
from jax.experimental import pallas as pl
from jax.experimental.pallas import tpu as pltpu
from jax.experimental.pallas import tpu_sc as plsc

N_NODES = 10000
N_EDGES = 320000
D_FEAT = 128
L = 16
NC, NS = 1, 16
NW = NC * NS

TILE = 128
N_TILES = N_EDGES // TILE
T_PER_W = N_TILES // NW
CHUNK = T_PER_W * TILE
REM_BASE = NW * CHUNK
N_REM = N_TILES - NW * T_PER_W
HALF = CHUNK // 2


def _tc_node_scores(b_ref, w_ref, h_ref, o_ref):
    rows = lax.broadcasted_iota(jnp.int32, (2, N_NODES), 0)
    bias = jnp.where(rows == 1, b_ref[0, 0], 0.0)
    o_ref[...] = (
        lax.dot_general(
            w_ref[...], h_ref[...],
            dimension_numbers=(((1,), (1,)), ((), ())),
            preferred_element_type=jnp.float32,
        )
        + bias
    )


_sc_mesh = plsc.VectorSubcoreMesh(
    core_axis_name="c", subcore_axis_name="s", num_cores=NC)


@functools.partial(
    pl.kernel,
    mesh=_sc_mesh,
    out_type=jax.ShapeDtypeStruct((1, N_EDGES), jnp.float32),
    compiler_params=pltpu.CompilerParams(needs_layout_passes=False),
    scratch_types=[
        pltpu.VMEM((N_NODES,), jnp.float32),
        pltpu.VMEM((N_NODES,), jnp.float32),
        pltpu.VMEM((2, CHUNK), jnp.int32),
        pltpu.VMEM((2, TILE), jnp.int32),
        pltpu.VMEM((CHUNK,), jnp.float32),
        pltpu.VMEM((TILE,), jnp.float32),
        pltpu.SemaphoreType.DMA,
        pltpu.SemaphoreType.DMA,
        pltpu.SemaphoreType.DMA,
    ],
)
def _sc_edge_gather(s_hbm, edge_hbm, out_hbm,
                    s1_v, s2_v, ei_v, ei2_v, out_v, out2_v,
                    sem, sem_b, sem_out):
    wid = lax.axis_index("s") * NC + lax.axis_index("c")
    base = wid * CHUNK
    rbase = REM_BASE + wid * TILE
    c1 = pltpu.async_copy(s_hbm.at[0], s1_v, sem)
    c2 = pltpu.async_copy(s_hbm.at[1], s2_v, sem)
    ca = pltpu.async_copy(edge_hbm.at[:, pl.ds(base, HALF)],
                          ei_v.at[:, pl.ds(0, HALF)], sem)
    cb = pltpu.async_copy(edge_hbm.at[:, pl.ds(base + HALF, HALF)],
                          ei_v.at[:, pl.ds(HALF, HALF)], sem_b)

    @pl.when(wid < N_REM)
    def _rem_fetch():
        pltpu.async_copy(edge_hbm.at[:, pl.ds(rbase, TILE)], ei2_v, sem_b)

    c1.wait()
    c2.wait()
    ca.wait()

    @plsc.parallel_loop(0, HALF, L, unroll=8)
    def _body_a(off):
        si = ei_v[0, pl.ds(off, L)]
        di = ei_v[1, pl.ds(off, L)]
        v1 = plsc.load_gather(s1_v, [si])
        v2 = plsc.load_gather(s2_v, [di])
        out_v[pl.ds(off, L)] = v1 + v2

    oa = pltpu.async_copy(out_v.at[pl.ds(0, HALF)],
                          out_hbm.at[0, pl.ds(base, HALF)], sem_out)
    cb.wait()

    @plsc.parallel_loop(HALF, CHUNK, L, unroll=8)
    def _body_b(off):
        si = ei_v[0, pl.ds(off, L)]
        di = ei_v[1, pl.ds(off, L)]
        v1 = plsc.load_gather(s1_v, [si])
        v2 = plsc.load_gather(s2_v, [di])
        out_v[pl.ds(off, L)] = v1 + v2

    ob = pltpu.async_copy(out_v.at[pl.ds(HALF, HALF)],
                          out_hbm.at[0, pl.ds(base + HALF, HALF)], sem_out)

    @pl.when(wid < N_REM)
    def _rem():
        pltpu.make_async_copy(edge_hbm.at[:, pl.ds(rbase, TILE)],
                              ei2_v, sem_b).wait()

        @plsc.parallel_loop(0, TILE, L, unroll=8)
        def _body2(off):
            si = ei2_v[0, pl.ds(off, L)]
            di = ei2_v[1, pl.ds(off, L)]
            v1 = plsc.load_gather(s1_v, [si])
            v2 = plsc.load_gather(s2_v, [di])
            out2_v[pl.ds(off, L)] = v1 + v2

        pltpu.sync_copy(out2_v, out_hbm.at[0, pl.ds(rbase, TILE)])

    oa.wait()
    ob.wait()


def kernel(h, edge_index, W, b):
    wmat = W.reshape(2, D_FEAT)
    s = pl.pallas_call(
        _tc_node_scores,
        in_specs=[
            pl.BlockSpec(memory_space=pltpu.SMEM),
            pl.BlockSpec((2, D_FEAT)),
            pl.BlockSpec((N_NODES, D_FEAT)),
        ],
        out_specs=pl.BlockSpec((2, N_NODES)),
        out_shape=jax.ShapeDtypeStruct((2, N_NODES), jnp.float32),
    )(b.reshape(1, 1), wmat, h)
    scores = _sc_edge_gather(s, edge_index)
    return scores.reshape(N_EDGES, 1)
